# baseline (speedup 1.0000x reference)
import functools

import jax
import jax.numpy as jnp
from jax import lax
from jax.experimental import pallas as pl
from jax.experimental.pallas import tpu as pltpu
from jax.experimental.pallas import tpu_sc as plsc

BEV_SIZE = 0.4
HGRID = 128
WGRID = 128
PC_RANGE = (0.0, 0.0, -3.0, 51.2, 51.2, 1.0)

BLK = 1024
NC, NS = 2, 16
NW = NC * NS
SC_ROWS = 256
SC_PASSES = 4
SC_CHUNK = 1024
NWORDS = BLK // 16
RING = 16


def _mlp_body(cnt_ref, xyzT_ref, feat_ref, W1fT_ref, W1r8_ref, s1_ref, b1_ref,
              W2T_ref, s2_ref, b2_ref, P16_ref, h_ref, idx_ref, bm_ref, *, bhw):
    x_min, y_min, z_min, _, _, z_max = PC_RANGE
    pid = pl.program_id(0)

    x = xyzT_ref[0:1, :]
    y = xyzT_ref[1:2, :]
    z = xyzT_ref[2:3, :]
    ix = jnp.clip(jnp.floor((x - x_min) / BEV_SIZE).astype(jnp.int32), 0, WGRID - 1)
    iy = jnp.clip(jnp.floor((y - y_min) / BEV_SIZE).astype(jnp.int32), 0, HGRID - 1)
    cx = x_min + (ix.astype(jnp.float32) + 0.5) * BEV_SIZE
    cy = y_min + (iy.astype(jnp.float32) + 0.5) * BEV_SIZE
    cz = 0.5 * (z_min + z_max)
    relx = x - cx
    rely = y - cy
    relz = z - cz

    n = pid * BLK + jax.lax.broadcasted_iota(jnp.int32, (1, BLK), 1)
    B = cnt_ref.shape[0]
    cum = cnt_ref[0]
    bat = (n >= cum).astype(jnp.int32)
    for b in range(1, B):
        cum = cum + cnt_ref[b]
        bat = bat + (n >= cum).astype(jnp.int32)
    bev = bat * (HGRID * WGRID) + iy * WGRID + ix
    bev = jnp.where(n < cum, bev, bhw)
    idx_ref[...] = bev.reshape(1, 1, BLK)

    bucket = bev // SC_ROWS
    iotac = jax.lax.broadcasted_iota(jnp.int32, (SC_PASSES * NW, 1), 0)
    oh = jnp.where(bucket == iotac, 1.0, 0.0)
    wbits = jnp.dot(oh, P16_ref[...], preferred_element_type=jnp.float32)
    bm_ref[...] = wbits.astype(jnp.int32).reshape(1, SC_PASSES * NW, NWORDS)

    zero = jnp.zeros_like(relx)
    pack = jnp.concatenate([relx, rely, relz, zero, zero, zero, zero, zero], axis=0)
    relT = pack.T

    h1 = jnp.dot(feat_ref[...], W1fT_ref[...], preferred_element_type=jnp.float32)
    h1 = h1 + jnp.dot(relT, W1r8_ref[...], preferred_element_type=jnp.float32)
    h1 = jnp.maximum(h1 * s1_ref[...] + b1_ref[...], 0.0)
    h2 = jnp.dot(h1, W2T_ref[...], preferred_element_type=jnp.float32)
    h2 = jnp.maximum(h2 * s2_ref[...] + b2_ref[...], 0.0)
    h_ref[...] = h2


def _sc_scatter_body(idx_hbm, h_hbm, bm_hbm, canvas_hbm, ibuf, bbuf, stage,
                     slab, mrow, gstate, sem_i, sem_b, sem_g):
    ci_c = lax.axis_index("c")
    ci_s = lax.axis_index("s")
    wid = ci_s * NC + ci_c
    iota16 = lax.broadcasted_iota(jnp.int32, (16,), 0)
    npad = idx_hbm.shape[0]
    nchunks = npad // SC_CHUNK

    def drain_one():
        dr = gstate[1]
        slot = lax.rem(dr, RING)
        pltpu.make_async_copy(h_hbm.at[pl.ds(0, 1)], stage.at[slot], sem_g).wait()
        row = mrow[slot]
        for t in range(16):
            a = slab[row, pl.ds(t * 16, 16)]
            b = stage[slot, 0, pl.ds(t * 16, 16)]
            slab[row, pl.ds(t * 16, 16)] = jnp.maximum(a, b)
        gstate[1] = dr + 1

    def fire_point(pid, row):
        @pl.when(gstate[0] - gstate[1] >= RING)
        def _():
            drain_one()
        g = gstate[0]
        slot = lax.rem(g, RING)
        mrow[slot] = row
        pltpu.make_async_copy(h_hbm.at[pl.ds(pid, 1)], stage.at[slot], sem_g).start()
        gstate[0] = g + 1

    for p in range(SC_PASSES):
        pw = p * NW + wid
        lo = pw * SC_ROWS
        hi = lo + SC_ROWS

        z16 = jnp.zeros((16,), jnp.float32)

        def zr(r, _):
            for t in range(16):
                slab[r, pl.ds(t * 16, 16)] = z16
            return 0
        lax.fori_loop(0, SC_ROWS, zr, 0)

        pltpu.make_async_copy(idx_hbm.at[pl.ds(0, SC_CHUNK)], ibuf.at[0], sem_i).start()
        pltpu.make_async_copy(idx_hbm.at[pl.ds(SC_CHUNK, SC_CHUNK)], ibuf.at[1], sem_i).start()
        pltpu.make_async_copy(bm_hbm.at[0, pw], bbuf.at[0], sem_b).start()
        pltpu.make_async_copy(bm_hbm.at[1, pw], bbuf.at[1], sem_b).start()

        def chunk_body(ci, _):
            slot2 = lax.rem(ci, 2)
            pltpu.make_async_copy(idx_hbm.at[pl.ds(0, SC_CHUNK)], ibuf.at[slot2], sem_i).wait()
            pltpu.make_async_copy(bm_hbm.at[0, pw], bbuf.at[slot2], sem_b).wait()

            for bw in range(NWORDS // 16):
                wv0 = bbuf[slot2, pl.ds(bw * 16, 16)]
                nzc = jnp.where(wv0 > 0, 1, 0)
                for d in (1, 2, 4, 8):
                    nzc = nzc + jnp.take(nzc, jnp.maximum(iota16 - d, 0)) * jnp.where(iota16 >= d, 1, 0)
                cw = nzc[15]

                @pl.when(cw > 0)
                def _(bw=bw, wv0=wv0, cw=cw):
                    def word_body(_w, wv):
                        wkey = jnp.where(wv > 0, iota16, 99)
                        for d in (1, 2, 4, 8):
                            wkey = jnp.minimum(wkey, jnp.take(wkey, jnp.minimum(iota16 + d, 15)))
                        j = wkey[0]
                        woff = bw * 16 + j
                        v = ibuf[slot2, pl.ds(woff * 16, 16)]
                        m = (v >= lo) & (v < hi)
                        c = jnp.where(m, 1, 0)
                        for d in (1, 2, 4, 8):
                            c = c + jnp.take(c, jnp.maximum(iota16 - d, 0)) * jnp.where(iota16 >= d, 1, 0)
                        cnt = c[15]

                        def bit_body(_t, mm):
                            key = jnp.where(mm > 0, (v - lo) * 16 + iota16, 99999)
                            for d in (1, 2, 4, 8):
                                key = jnp.minimum(key, jnp.take(key, jnp.minimum(iota16 + d, 15)))
                            k0 = key[0]
                            l = lax.rem(k0, 16)
                            fire_point(ci * SC_CHUNK + woff * 16 + l, k0 // 16)
                            return jnp.where(iota16 == l, 0, mm)
                        lax.fori_loop(0, cnt, bit_body, jnp.where(m, 1, 0))
                        return jnp.where(iota16 == j, 0, wv)
                    lax.fori_loop(0, cw, word_body, wv0)

            @pl.when(ci + 2 < nchunks)
            def _():
                pltpu.make_async_copy(
                    idx_hbm.at[pl.ds((ci + 2) * SC_CHUNK, SC_CHUNK)],
                    ibuf.at[slot2], sem_i).start()
                pltpu.make_async_copy(bm_hbm.at[ci + 2, pw], bbuf.at[slot2], sem_b).start()
            return 0

        lax.fori_loop(0, nchunks, chunk_body, 0)

        ndrain = gstate[0] - gstate[1]

        def dr_all(_i, _c):
            drain_one()
            return _c
        lax.fori_loop(0, ndrain, dr_all, 0)
        gstate[0] = 0
        gstate[1] = 0

        pltpu.sync_copy(slab, canvas_hbm.at[pl.ds(lo, SC_ROWS)])


@jax.jit
def kernel(xyz, xyz_batch_cnt, point_features, W1, g1, b1, W2, g2, b2):
    N, C = point_features.shape
    B = xyz_batch_cnt.shape[0]
    bhw = B * HGRID * WGRID
    nblocks = -(-N // BLK)
    npad = nblocks * BLK

    xyzT = jnp.pad(xyz, ((0, npad - N), (0, 0))).T
    feat = jnp.pad(point_features, ((0, npad - N), (0, 0)))

    inv = 1.0 / jnp.sqrt(jnp.float32(1.0 + 1e-5))
    W1fT = W1[:, 3:].T
    W1r8 = jnp.pad(W1[:, :3].T, ((0, 5), (0, 0)))
    s1 = (inv * g1).reshape(1, -1)
    b1r = b1.reshape(1, -1)
    W2T = W2.T
    s2 = (inv * g2).reshape(1, -1)
    b2r = b2.reshape(1, -1)

    ii = jnp.arange(BLK)
    P16 = ((ii[:, None] // 16 == jnp.arange(NWORDS)[None, :])
           .astype(jnp.float32) * (2.0 ** (ii % 16)).astype(jnp.float32)[:, None])

    D = W2.shape[0]
    PWS = SC_PASSES * NW
    h, idx3, bm = pl.pallas_call(
        functools.partial(_mlp_body, bhw=bhw),
        grid=(nblocks,),
        in_specs=[
            pl.BlockSpec(memory_space=pltpu.SMEM),
            pl.BlockSpec((3, BLK), lambda i: (0, i)),
            pl.BlockSpec((BLK, C), lambda i: (i, 0)),
            pl.BlockSpec((C, 128), lambda i: (0, 0)),
            pl.BlockSpec((8, 128), lambda i: (0, 0)),
            pl.BlockSpec((1, 128), lambda i: (0, 0)),
            pl.BlockSpec((1, 128), lambda i: (0, 0)),
            pl.BlockSpec((128, D), lambda i: (0, 0)),
            pl.BlockSpec((1, D), lambda i: (0, 0)),
            pl.BlockSpec((1, D), lambda i: (0, 0)),
            pl.BlockSpec((BLK, NWORDS), lambda i: (0, 0)),
        ],
        out_specs=[
            pl.BlockSpec((BLK, D), lambda i: (i, 0)),
            pl.BlockSpec((1, 1, BLK), lambda i: (i, 0, 0)),
            pl.BlockSpec((1, PWS, NWORDS), lambda i: (i, 0, 0)),
        ],
        out_shape=[
            jax.ShapeDtypeStruct((npad, D), jnp.float32),
            jax.ShapeDtypeStruct((nblocks, 1, BLK), jnp.int32),
            jax.ShapeDtypeStruct((nblocks, PWS, NWORDS), jnp.int32),
        ],
    )(xyz_batch_cnt, xyzT, feat, W1fT, W1r8, s1, b1r, W2T, s2, b2r, P16)

    idx_flat = idx3.reshape(npad)
    crows = SC_PASSES * NW * SC_ROWS

    mesh = plsc.VectorSubcoreMesh(core_axis_name="c", subcore_axis_name="s")
    canvas = pl.kernel(
        _sc_scatter_body,
        mesh=mesh,
        out_type=jax.ShapeDtypeStruct((crows, D), jnp.float32),
        scratch_types=[
            pltpu.VMEM((2, SC_CHUNK), jnp.int32),
            pltpu.VMEM((2, NWORDS), jnp.int32),
            pltpu.VMEM((RING, 1, D), jnp.float32),
            pltpu.VMEM((SC_ROWS, D), jnp.float32),
            pltpu.SMEM((RING,), jnp.int32),
            pltpu.SMEM((2,), jnp.int32),
            pltpu.SemaphoreType.DMA,
            pltpu.SemaphoreType.DMA,
            pltpu.SemaphoreType.DMA,
        ],
    )(idx_flat, h, bm)

    canvas = canvas[:bhw]
    return canvas.reshape(B, HGRID, WGRID, D).transpose(0, 3, 1, 2)


# speedup vs baseline: 1.0569x; 1.0569x over previous
import functools

import jax
import jax.numpy as jnp
from jax import lax
from jax.experimental import pallas as pl
from jax.experimental.pallas import tpu as pltpu
from jax.experimental.pallas import tpu_sc as plsc

BEV_SIZE = 0.4
HGRID = 128
WGRID = 128
PC_RANGE = (0.0, 0.0, -3.0, 51.2, 51.2, 1.0)

BLK = 1024
NC, NS = 2, 16
NW = NC * NS
SC_ROWS = 256
SC_PASSES = 4
SC_CHUNK = 1024
NWORDS = BLK // 16
RING = 16


def _mlp_body(cnt_ref, xyzT_ref, feat_ref, W1fT_ref, W1r8_ref, s1_ref, b1_ref,
              W2T_ref, s2_ref, b2_ref, P16_ref, h_ref, idx_ref, bm_ref, *, bhw):
    x_min, y_min, z_min, _, _, z_max = PC_RANGE
    pid = pl.program_id(0)

    x = xyzT_ref[0:1, :]
    y = xyzT_ref[1:2, :]
    z = xyzT_ref[2:3, :]
    ix = jnp.clip(jnp.floor((x - x_min) / BEV_SIZE).astype(jnp.int32), 0, WGRID - 1)
    iy = jnp.clip(jnp.floor((y - y_min) / BEV_SIZE).astype(jnp.int32), 0, HGRID - 1)
    cx = x_min + (ix.astype(jnp.float32) + 0.5) * BEV_SIZE
    cy = y_min + (iy.astype(jnp.float32) + 0.5) * BEV_SIZE
    cz = 0.5 * (z_min + z_max)
    relx = x - cx
    rely = y - cy
    relz = z - cz

    n = pid * BLK + jax.lax.broadcasted_iota(jnp.int32, (1, BLK), 1)
    B = cnt_ref.shape[0]
    cum = cnt_ref[0]
    bat = (n >= cum).astype(jnp.int32)
    for b in range(1, B):
        cum = cum + cnt_ref[b]
        bat = bat + (n >= cum).astype(jnp.int32)
    bev = bat * (HGRID * WGRID) + iy * WGRID + ix
    bev = jnp.where(n < cum, bev, bhw)
    idx_ref[...] = bev.reshape(1, 1, BLK)

    bucket = bev // SC_ROWS
    iotac = jax.lax.broadcasted_iota(jnp.int32, (SC_PASSES * NW, 1), 0)
    oh = jnp.where(bucket == iotac, 1.0, 0.0)
    wbits = jnp.dot(oh, P16_ref[...], preferred_element_type=jnp.float32)
    bm_ref[...] = wbits.astype(jnp.int32).reshape(1, SC_PASSES * NW, NWORDS)

    zero = jnp.zeros_like(relx)
    pack = jnp.concatenate([relx, rely, relz, zero, zero, zero, zero, zero], axis=0)
    relT = pack.T

    h1 = jnp.dot(feat_ref[...], W1fT_ref[...], preferred_element_type=jnp.float32)
    h1 = h1 + jnp.dot(relT, W1r8_ref[...], preferred_element_type=jnp.float32)
    h1 = jnp.maximum(h1 * s1_ref[...] + b1_ref[...], 0.0)
    h2 = jnp.dot(h1, W2T_ref[...], preferred_element_type=jnp.float32)
    h2 = jnp.maximum(h2 * s2_ref[...] + b2_ref[...], 0.0)
    h_ref[...] = h2


def _sc_scatter_body(idx_hbm, h_hbm, bm_hbm, canvas_hbm, ibuf, bbuf, stage,
                     slab, mrow, gstate, sem_i, sem_b, sem_g):
    ci_c = lax.axis_index("c")
    ci_s = lax.axis_index("s")
    wid = ci_s * NC + ci_c
    iota16 = lax.broadcasted_iota(jnp.int32, (16,), 0)
    npad = idx_hbm.shape[0]
    nchunks = npad // SC_CHUNK

    def drain_one():
        dr = gstate[1]
        slot = lax.rem(dr, RING)
        pltpu.make_async_copy(h_hbm.at[pl.ds(0, 1)], stage.at[slot], sem_g).wait()
        row = mrow[slot]
        for t in range(16):
            a = slab[row, pl.ds(t * 16, 16)]
            b = stage[slot, 0, pl.ds(t * 16, 16)]
            slab[row, pl.ds(t * 16, 16)] = jnp.maximum(a, b)
        gstate[1] = dr + 1

    def fire_point(pid, row):
        @pl.when(gstate[0] - gstate[1] >= RING)
        def _():
            drain_one()
        g = gstate[0]
        slot = lax.rem(g, RING)
        mrow[slot] = row
        pltpu.make_async_copy(h_hbm.at[pl.ds(pid, 1)], stage.at[slot], sem_g).start()
        gstate[0] = g + 1

    for p in range(SC_PASSES):
        pw = p * NW + wid
        lo = pw * SC_ROWS
        hi = lo + SC_ROWS

        z16 = jnp.zeros((16,), jnp.float32)

        def zr(r, _):
            for t in range(16):
                slab[r, pl.ds(t * 16, 16)] = z16
            return 0
        lax.fori_loop(0, SC_ROWS, zr, 0)

        pltpu.make_async_copy(idx_hbm.at[pl.ds(0, SC_CHUNK)], ibuf.at[0], sem_i).start()
        pltpu.make_async_copy(idx_hbm.at[pl.ds(SC_CHUNK, SC_CHUNK)], ibuf.at[1], sem_i).start()
        pltpu.make_async_copy(bm_hbm.at[0, pw], bbuf.at[0], sem_b).start()
        pltpu.make_async_copy(bm_hbm.at[1, pw], bbuf.at[1], sem_b).start()

        def chunk_body(ci, _):
            slot2 = lax.rem(ci, 2)
            pltpu.make_async_copy(idx_hbm.at[pl.ds(0, SC_CHUNK)], ibuf.at[slot2], sem_i).wait()
            pltpu.make_async_copy(bm_hbm.at[0, pw], bbuf.at[slot2], sem_b).wait()

            for bw in range(NWORDS // 16):
                wv0 = bbuf[slot2, pl.ds(bw * 16, 16)]
                nzc = jnp.where(wv0 > 0, 1, 0)
                for d in (1, 2, 4, 8):
                    nzc = nzc + jnp.take(nzc, jnp.maximum(iota16 - d, 0)) * jnp.where(iota16 >= d, 1, 0)
                cw = nzc[15]

                @pl.when(cw > 0)
                def _(bw=bw, wv0=wv0, cw=cw):
                    def word_body(_w, wv):
                        wkey = jnp.where(wv > 0, iota16 * 4194304 + wv, 2000000000)
                        for d in (1, 2, 4, 8):
                            wkey = jnp.minimum(wkey, jnp.take(wkey, jnp.minimum(iota16 + d, 15)))
                        k0 = wkey[0]
                        j = k0 // 4194304
                        cnt = lax.rem(k0, 4194304) // 65536
                        woff = bw * 16 + j
                        v = ibuf[slot2, pl.ds(woff * 16, 16)]
                        m = (v >= lo) & (v < hi)

                        def bit_body(_t, mm):
                            key = jnp.where(mm > 0, (v - lo) * 16 + iota16, 99999)
                            for d in (1, 2, 4, 8):
                                key = jnp.minimum(key, jnp.take(key, jnp.minimum(iota16 + d, 15)))
                            k0 = key[0]
                            l = lax.rem(k0, 16)
                            fire_point(ci * SC_CHUNK + woff * 16 + l, k0 // 16)
                            return jnp.where(iota16 == l, 0, mm)
                        lax.fori_loop(0, cnt, bit_body, jnp.where(m, 1, 0))
                        return jnp.where(iota16 == j, 0, wv)
                    lax.fori_loop(0, cw, word_body, wv0)

            @pl.when(ci + 2 < nchunks)
            def _():
                pltpu.make_async_copy(
                    idx_hbm.at[pl.ds((ci + 2) * SC_CHUNK, SC_CHUNK)],
                    ibuf.at[slot2], sem_i).start()
                pltpu.make_async_copy(bm_hbm.at[ci + 2, pw], bbuf.at[slot2], sem_b).start()
            return 0

        lax.fori_loop(0, nchunks, chunk_body, 0)

        ndrain = gstate[0] - gstate[1]

        def dr_all(_i, _c):
            drain_one()
            return _c
        lax.fori_loop(0, ndrain, dr_all, 0)
        gstate[0] = 0
        gstate[1] = 0

        pltpu.sync_copy(slab, canvas_hbm.at[pl.ds(lo, SC_ROWS)])


@jax.jit
def kernel(xyz, xyz_batch_cnt, point_features, W1, g1, b1, W2, g2, b2):
    N, C = point_features.shape
    B = xyz_batch_cnt.shape[0]
    bhw = B * HGRID * WGRID
    nblocks = -(-N // BLK)
    npad = nblocks * BLK

    xyzT = jnp.pad(xyz, ((0, npad - N), (0, 0))).T
    feat = jnp.pad(point_features, ((0, npad - N), (0, 0)))

    inv = 1.0 / jnp.sqrt(jnp.float32(1.0 + 1e-5))
    W1fT = W1[:, 3:].T
    W1r8 = jnp.pad(W1[:, :3].T, ((0, 5), (0, 0)))
    s1 = (inv * g1).reshape(1, -1)
    b1r = b1.reshape(1, -1)
    W2T = W2.T
    s2 = (inv * g2).reshape(1, -1)
    b2r = b2.reshape(1, -1)

    ii = jnp.arange(BLK)
    P16 = ((ii[:, None] // 16 == jnp.arange(NWORDS)[None, :])
           .astype(jnp.float32)
           * ((2.0 ** (ii % 16)) + 65536.0).astype(jnp.float32)[:, None])

    D = W2.shape[0]
    PWS = SC_PASSES * NW
    h, idx3, bm = pl.pallas_call(
        functools.partial(_mlp_body, bhw=bhw),
        grid=(nblocks,),
        in_specs=[
            pl.BlockSpec(memory_space=pltpu.SMEM),
            pl.BlockSpec((3, BLK), lambda i: (0, i)),
            pl.BlockSpec((BLK, C), lambda i: (i, 0)),
            pl.BlockSpec((C, 128), lambda i: (0, 0)),
            pl.BlockSpec((8, 128), lambda i: (0, 0)),
            pl.BlockSpec((1, 128), lambda i: (0, 0)),
            pl.BlockSpec((1, 128), lambda i: (0, 0)),
            pl.BlockSpec((128, D), lambda i: (0, 0)),
            pl.BlockSpec((1, D), lambda i: (0, 0)),
            pl.BlockSpec((1, D), lambda i: (0, 0)),
            pl.BlockSpec((BLK, NWORDS), lambda i: (0, 0)),
        ],
        out_specs=[
            pl.BlockSpec((BLK, D), lambda i: (i, 0)),
            pl.BlockSpec((1, 1, BLK), lambda i: (i, 0, 0)),
            pl.BlockSpec((1, PWS, NWORDS), lambda i: (i, 0, 0)),
        ],
        out_shape=[
            jax.ShapeDtypeStruct((npad, D), jnp.float32),
            jax.ShapeDtypeStruct((nblocks, 1, BLK), jnp.int32),
            jax.ShapeDtypeStruct((nblocks, PWS, NWORDS), jnp.int32),
        ],
    )(xyz_batch_cnt, xyzT, feat, W1fT, W1r8, s1, b1r, W2T, s2, b2r, P16)

    idx_flat = idx3.reshape(npad)
    crows = SC_PASSES * NW * SC_ROWS

    mesh = plsc.VectorSubcoreMesh(core_axis_name="c", subcore_axis_name="s")
    canvas = pl.kernel(
        _sc_scatter_body,
        mesh=mesh,
        out_type=jax.ShapeDtypeStruct((crows, D), jnp.float32),
        scratch_types=[
            pltpu.VMEM((2, SC_CHUNK), jnp.int32),
            pltpu.VMEM((2, NWORDS), jnp.int32),
            pltpu.VMEM((RING, 1, D), jnp.float32),
            pltpu.VMEM((SC_ROWS, D), jnp.float32),
            pltpu.SMEM((RING,), jnp.int32),
            pltpu.SMEM((2,), jnp.int32),
            pltpu.SemaphoreType.DMA,
            pltpu.SemaphoreType.DMA,
            pltpu.SemaphoreType.DMA,
        ],
    )(idx_flat, h, bm)

    canvas = canvas[:bhw]
    return canvas.reshape(B, HGRID, WGRID, D).transpose(0, 3, 1, 2)
